# baseline (speedup 1.0000x reference)
import functools

import jax
import jax.numpy as jnp
from jax import lax
from jax.experimental import pallas as pl
from jax.experimental.pallas import tpu as pltpu

_K = 7
_NH = 3
_GC = 32
_H = 64
_W = 64
_TR = 8


def _conv1x1_kernel(x_ref, w_ref, b_ref, o_ref):
    x = x_ref[0]
    w = w_ref[...]
    y = jnp.dot(w, x, preferred_element_type=jnp.float32)
    o_ref[0] = y + b_ref[...]


def _conv1x1_res_kernel(x_ref, r_ref, w_ref, b_ref, o_ref):
    x = x_ref[0] + r_ref[0]
    w = w_ref[...]
    y = jnp.dot(w, x, preferred_element_type=jnp.float32)
    o_ref[0] = y + b_ref[...]


def _conv1x1(x, w, b, residual=None, *, tile=1024):
    B, C_in, S = x.shape
    C_out = w.shape[0]
    grid = (B, S // tile)

    x_spec = pl.BlockSpec((1, C_in, tile), lambda bi, si: (bi, 0, si))
    w_spec = pl.BlockSpec((C_out, C_in), lambda bi, si: (0, 0))
    b_spec = pl.BlockSpec((C_out, 1), lambda bi, si: (0, 0))
    o_spec = pl.BlockSpec((1, C_out, tile), lambda bi, si: (bi, 0, si))
    b2 = b.reshape(C_out, 1)

    if residual is None:
        kern = _conv1x1_kernel
        operands = (x, w, b2)
        in_specs = [x_spec, w_spec, b_spec]
    else:
        kern = _conv1x1_res_kernel
        operands = (x, residual, w, b2)
        in_specs = [x_spec, x_spec, w_spec, b_spec]

    return pl.pallas_call(
        kern,
        out_shape=jax.ShapeDtypeStruct((B, C_out, S), x.dtype),
        grid=grid,
        in_specs=in_specs,
        out_specs=o_spec,
        compiler_params=pltpu.CompilerParams(
            dimension_semantics=("parallel", "parallel")),
    )(*operands)


_W2 = 2 * _W


def _cshift_c(sref, dx, a, b):
    n = b - a
    src = sref[a:b]
    x = pltpu.roll(src, (_NH - dx) % _W2, axis=2)
    il = lax.broadcasted_iota(jnp.int32, (1, _H, _W2), 2)
    m1 = (il % _W) == (_NH - 1)
    x = jnp.where(m1, pltpu.roll(x, _W2 - 1, axis=2), x)
    m2 = (il % _W) < (_NH - 1)
    x = jnp.where(m2, pltpu.roll(x, _W2 - 2, axis=2), x)
    m3 = (il % _W) == (_W - _NH)
    x = jnp.where(m3, pltpu.roll(x, 1, axis=2), x)
    m4 = (il % _W) > (_W - _NH)
    x = jnp.where(m4, pltpu.roll(x, 2, axis=2), x)
    return x


def _shift_rows2d(a, sh, rows, w):
    if sh == 0:
        return a
    return pltpu.roll(a, (-sh) % rows, axis=a.ndim - 2)


def _rows_p(sref, dx, dy, si):
    r0 = si * _TR
    if si == 0:
        top = jnp.broadcast_to(sref[dx, :, pl.ds(dy, 1), :],
                               (_GC, _NH + 1, _W2))
        rest = sref[dx, :, pl.ds(dy + 1, _TR - _NH - 1), :]
        return jnp.concatenate([top, rest], axis=1)
    body = sref[dx, :, pl.ds(r0 - _NH + dy, _TR - _NH), :]
    bot = jnp.broadcast_to(sref[dx, :, pl.ds(_H - _K + dy, 1), :],
                           (_GC, _NH, _W2))
    return jnp.concatenate([body, bot], axis=1)


def _na_kernel(q_ref, k_ref, v_ref, o_ref, qs3, kvs, ls, asc, oacc, *, scale):
    KK = _K * _K
    CH = 16
    chunks = [(c, c + CH) for c in range(0, _GC, CH)]
    SI_LAST = (_H // _TR) - 1

    for a, b in chunks:
        qs3[a:b] = jnp.concatenate(
            [q_ref[0, 0, a:b], q_ref[0, 1, a:b]], axis=2) * scale

    for a, b in chunks:
        asc[a:b] = jnp.concatenate(
            [k_ref[0, 0, 0, a:b], k_ref[0, 0, 1, a:b]], axis=2)
    for dx in range(_K):
        for a, b in chunks:
            kvs[dx, a:b] = _cshift_c(asc, dx, a, b)

    m = jnp.full((_H, _W2), -jnp.inf, dtype=jnp.float32)
    for dy in range(_K):
        sh = _NH - dy
        for a, b in chunks:
            asc[a:b] = _shift_rows2d(qs3[a:b], sh, _H, _W2)

        def p1(dx, m, dy=dy, sh=sh):
            lu = jnp.zeros((_H, _W2), dtype=jnp.float32)
            for a, b in chunks:
                lu = lu + jnp.sum(asc[a:b] * kvs[dx, a:b], axis=0)
            lg = _shift_rows2d(lu, -sh, _H, _W2)
            ls[dy * _K + dx] = lg
            return jnp.maximum(m, lg)

        m = lax.fori_loop(0, _K, p1, m, unroll=True)

    me_all = []
    for si in (0, SI_LAST):
        r0 = si * _TR
        qsv = qs3[:, r0:r0 + _TR, :]

        def e1(o, me, si=si, qsv=qsv, r0=r0):
            dy = o // _K
            dx = o - dy * _K
            kp = _rows_p(kvs, dx, dy, si)
            lg = jnp.sum(qsv * kp, axis=0)
            ls[o, r0:r0 + _TR] = lg
            return jnp.maximum(me, lg)

        me_all.append(lax.fori_loop(
            0, KK, e1, jnp.full((_TR, _W2), -jnp.inf, dtype=jnp.float32),
            unroll=7))

    for a, b in chunks:
        asc[a:b] = jnp.concatenate(
            [v_ref[0, 0, 0, a:b], v_ref[0, 0, 1, a:b]], axis=2)
    for dx in range(_K):
        for a, b in chunks:
            kvs[dx, a:b] = _cshift_c(asc, dx, a, b)

    den = jnp.zeros((_H, _W2), dtype=jnp.float32)
    for a, b in chunks:
        oacc[a:b] = jnp.zeros((CH, _H, _W2), dtype=jnp.float32)
    for dy in range(_K):
        sh = _NH - dy
        for a, b in chunks:
            asc[a:b] = jnp.zeros((CH, _H, _W2), dtype=jnp.float32)

        def p2(dx, den, dy=dy, sh=sh):
            p = jnp.exp(ls[dy * _K + dx] - m)
            pt = _shift_rows2d(p, sh, _H, _W2)
            for a, b in chunks:
                asc[a:b] = asc[a:b] + pt[None] * kvs[dx, a:b]
            return den + p

        den = lax.fori_loop(0, _K, p2, den, unroll=True)
        for a, b in chunks:
            oacc[a:b] = oacc[a:b] + _shift_rows2d(asc[a:b], -sh, _H, _W2)

    inv = pl.reciprocal(den, approx=False)
    for a, b in chunks:
        res = oacc[a:b] * inv[None]
        o_ref[0, 0, a:b] = res[:, :, :_W]
        o_ref[0, 1, a:b] = res[:, :, _W:]

    for si, me in zip((0, SI_LAST), me_all):
        r0 = si * _TR

        def e2(o, carry, si=si, r0=r0, me=me):
            dene, acce = carry
            dy = o // _K
            dx = o - dy * _K
            p = jnp.exp(ls[o, r0:r0 + _TR] - me)
            vp = _rows_p(kvs, dx, dy, si)
            return dene + p, acce + p[None] * vp

        dene, acce = lax.fori_loop(
            0, KK, e2,
            (jnp.zeros((_TR, _W2), dtype=jnp.float32),
             jnp.zeros((_GC, _TR, _W2), dtype=jnp.float32)), unroll=7)

        inve = pl.reciprocal(dene, approx=False)
        rese = acce * inve[None]
        o_ref[0, 0, :, r0:r0 + _TR, :] = rese[:, :, :_W]
        o_ref[0, 1, :, r0:r0 + _TR, :] = rese[:, :, _W:]


def _na2d(q, kv, *, scale):
    B, G = q.shape[0], q.shape[1]
    kern = functools.partial(_na_kernel, scale=scale)
    return pl.pallas_call(
        kern,
        out_shape=jax.ShapeDtypeStruct(q.shape, q.dtype),
        grid=(B, G // 2),
        in_specs=[
            pl.BlockSpec((1, 2, _GC, _H, _W), lambda bi, pi: (bi, pi, 0, 0, 0)),
            pl.BlockSpec((1, 1, 2, _GC, _H, _W),
                         lambda bi, pi: (bi, 0, pi, 0, 0, 0)),
            pl.BlockSpec((1, 1, 2, _GC, _H, _W),
                         lambda bi, pi: (bi, 1, pi, 0, 0, 0)),
        ],
        out_specs=pl.BlockSpec((1, 2, _GC, _H, _W),
                               lambda bi, pi: (bi, pi, 0, 0, 0)),
        scratch_shapes=[
            pltpu.VMEM((_GC, _H, _W2), jnp.float32),
            pltpu.VMEM((_K, _GC, _H, _W2), jnp.float32),
            pltpu.VMEM((_K * _K, _H, _W2), jnp.float32),
            pltpu.VMEM((_GC, _H, _W2), jnp.float32),
            pltpu.VMEM((_GC, _H, _W2), jnp.float32),
        ],
        compiler_params=pltpu.CompilerParams(
            dimension_semantics=("parallel", "parallel")),
    )(q, kv, kv)


def _sample_kernel(x_ref, off_ref, o_ref):
    H, W = _H, _W
    o = off_ref[0, 0]
    offy = jnp.tanh(o[0]) * jnp.float32(1.0 / (H - 1))
    offx = jnp.tanh(o[1]) * jnp.float32(1.0 / (W - 1))
    iy = jax.lax.broadcasted_iota(jnp.int32, (H, W), 0).astype(jnp.float32)
    ix = jax.lax.broadcasted_iota(jnp.int32, (H, W), 1).astype(jnp.float32)
    ref_y = (iy + 0.5) / (H - 1.0) * 2.0 - 1.0
    ref_x = (ix + 0.5) / (W - 1.0) * 2.0 - 1.0
    gy = (offy + ref_y + 1.0) * 0.5 * (H - 1)
    gx = (offx + ref_x + 1.0) * 0.5 * (W - 1)
    wy1 = gy - iy
    wy0 = 1.0 - wy1
    wx1 = gx - ix
    wx0 = 1.0 - wx1

    xx = x_ref[0, 0]
    zc = jnp.zeros((_GC, H, 1), dtype=jnp.float32)
    zr = jnp.zeros((_GC, 1, W), dtype=jnp.float32)
    x_e = jnp.concatenate([xx[:, :, 1:], zc], axis=2)
    x_s = jnp.concatenate([xx[:, 1:, :], zr], axis=1)
    x_se = jnp.concatenate([x_e[:, 1:, :], zr], axis=1)

    out = (xx * (wy0 * wx0)[None] + x_e * (wy0 * wx1)[None]
           + x_s * (wy1 * wx0)[None] + x_se * (wy1 * wx1)[None])
    o_ref[0, 0] = out


def _deform_sample(x_g, off_raw):
    B, G = x_g.shape[0], x_g.shape[1]
    return pl.pallas_call(
        _sample_kernel,
        out_shape=jax.ShapeDtypeStruct(x_g.shape, x_g.dtype),
        grid=(B, G),
        in_specs=[
            pl.BlockSpec((1, 1, _GC, _H, _W), lambda bi, gi: (bi, gi, 0, 0, 0)),
            pl.BlockSpec((1, 1, 2, _H, _W), lambda bi, gi: (bi, gi, 0, 0, 0)),
        ],
        out_specs=pl.BlockSpec((1, 1, _GC, _H, _W),
                               lambda bi, gi: (bi, gi, 0, 0, 0)),
        compiler_params=pltpu.CompilerParams(
            dimension_semantics=("parallel", "parallel")),
    )(x_g, off_raw)


def _depthwise_conv(x, w, b, *, stride=1, padding=0):
    C = x.shape[1]
    y = lax.conv_general_dilated(
        x, w, window_strides=(stride, stride),
        padding=[(padding, padding), (padding, padding)],
        dimension_numbers=("NCHW", "OIHW", "NCHW"),
        feature_group_count=C)
    if b is not None:
        y = y + b[None, :, None, None]
    return y


def _layernorm2d(x, gamma, beta, eps=1e-6):
    u = jnp.mean(x, axis=1, keepdims=True)
    s = jnp.mean((x - u) ** 2, axis=1, keepdims=True)
    xn = (x - u) / jnp.sqrt(s + eps)
    return gamma[None, :, None, None] * xn + beta[None, :, None, None]


def kernel(wq, bq, wk, bk, wv, bv, wo, bo, off_dw_w, off_dw_b,
           off_ln_g, off_ln_b, off_pw_w, rpe_w, rpe_b, x):
    num_heads = 4
    offset_range_factor = 1.0
    B, C, H, W = x.shape
    G = num_heads
    gc = C // G
    scale = gc ** (-0.5)
    K = _K
    S = H * W

    x_flat = x.reshape(B, C, S)

    q = _conv1x1(x_flat, wq, bq)
    q_img = q.reshape(B, C, H, W)

    q_off = q_img.reshape(B * G, gc, H, W)
    t = _depthwise_conv(q_off, off_dw_w, off_dw_b, stride=1, padding=K // 2)
    t = _layernorm2d(t, off_ln_g, off_ln_b)
    t = jax.nn.gelu(t, approximate=False)
    off_raw = jnp.einsum("oc,bchw->bohw", off_pw_w, t)

    x_sampled = _deform_sample(x.reshape(B, G, gc, H, W),
                               off_raw.reshape(B, G, 2, H, W))
    x_sampled = x_sampled.reshape(B, C, S)

    lepe = _depthwise_conv(q_img, rpe_w, rpe_b, stride=1, padding=1)
    lepe_flat = lepe.reshape(B, C, S)

    wkv = jnp.concatenate([wk, wv], axis=0)
    bkv = jnp.concatenate([bk, bv], axis=0)
    kv = _conv1x1(x_sampled, wkv, bkv)

    q_g = q.reshape(B, G, gc, H, W)
    kv_g = kv.reshape(B, 2, G, gc, H, W)
    out = _na2d(q_g, kv_g, scale=scale)
    out = out.reshape(B, C, S)

    y = _conv1x1(out, wo, bo, residual=lepe_flat)
    return y.reshape(B, C, H, W)


# speedup vs baseline: 1.0120x; 1.0120x over previous
import functools

import jax
import jax.numpy as jnp
from jax import lax
from jax.experimental import pallas as pl
from jax.experimental.pallas import tpu as pltpu

_K = 7
_NH = 3
_GC = 32
_H = 64
_W = 64
_TR = 8


def _conv1x1_kernel(x_ref, w_ref, b_ref, o_ref):
    x = x_ref[0]
    w = w_ref[...]
    y = jnp.dot(w, x, preferred_element_type=jnp.float32)
    o_ref[0] = y + b_ref[...]


def _conv1x1_res_kernel(x_ref, r_ref, w_ref, b_ref, o_ref):
    x = x_ref[0] + r_ref[0]
    w = w_ref[...]
    y = jnp.dot(w, x, preferred_element_type=jnp.float32)
    o_ref[0] = y + b_ref[...]


def _conv1x1(x, w, b, residual=None, *, tile=1024):
    B, C_in, S = x.shape
    C_out = w.shape[0]
    grid = (B, S // tile)

    x_spec = pl.BlockSpec((1, C_in, tile), lambda bi, si: (bi, 0, si))
    w_spec = pl.BlockSpec((C_out, C_in), lambda bi, si: (0, 0))
    b_spec = pl.BlockSpec((C_out, 1), lambda bi, si: (0, 0))
    o_spec = pl.BlockSpec((1, C_out, tile), lambda bi, si: (bi, 0, si))
    b2 = b.reshape(C_out, 1)

    if residual is None:
        kern = _conv1x1_kernel
        operands = (x, w, b2)
        in_specs = [x_spec, w_spec, b_spec]
    else:
        kern = _conv1x1_res_kernel
        operands = (x, residual, w, b2)
        in_specs = [x_spec, x_spec, w_spec, b_spec]

    return pl.pallas_call(
        kern,
        out_shape=jax.ShapeDtypeStruct((B, C_out, S), x.dtype),
        grid=grid,
        in_specs=in_specs,
        out_specs=o_spec,
        compiler_params=pltpu.CompilerParams(
            dimension_semantics=("parallel", "parallel")),
    )(*operands)


_W2 = 2 * _W


def _cshift_c(sref, dx, a, b):
    n = b - a
    src = sref[a:b]
    x = pltpu.roll(src, (_NH - dx) % _W2, axis=2)
    il = lax.broadcasted_iota(jnp.int32, (1, _H, _W2), 2)
    m1 = (il % _W) == (_NH - 1)
    x = jnp.where(m1, pltpu.roll(x, _W2 - 1, axis=2), x)
    m2 = (il % _W) < (_NH - 1)
    x = jnp.where(m2, pltpu.roll(x, _W2 - 2, axis=2), x)
    m3 = (il % _W) == (_W - _NH)
    x = jnp.where(m3, pltpu.roll(x, 1, axis=2), x)
    m4 = (il % _W) > (_W - _NH)
    x = jnp.where(m4, pltpu.roll(x, 2, axis=2), x)
    return x


def _shift_rows2d(a, sh, rows, w):
    if sh == 0:
        return a
    return pltpu.roll(a, (-sh) % rows, axis=a.ndim - 2)


def _rows_p(sref, dx, dy, si):
    r0 = si * _TR
    if si == 0:
        top = jnp.broadcast_to(sref[dx, :, pl.ds(dy, 1), :],
                               (_GC, _NH + 1, _W2))
        rest = sref[dx, :, pl.ds(dy + 1, _TR - _NH - 1), :]
        return jnp.concatenate([top, rest], axis=1)
    body = sref[dx, :, pl.ds(r0 - _NH + dy, _TR - _NH), :]
    bot = jnp.broadcast_to(sref[dx, :, pl.ds(_H - _K + dy, 1), :],
                           (_GC, _NH, _W2))
    return jnp.concatenate([body, bot], axis=1)


def _na_kernel(q_ref, k_ref, v_ref, o_ref, qs3, kvs, ls, asc, oacc, *, scale):
    KK = _K * _K
    CH = 8
    chunks = [(c, c + CH) for c in range(0, _GC, CH)]
    SI_LAST = (_H // _TR) - 1

    for a, b in chunks:
        qs3[a:b] = jnp.concatenate(
            [q_ref[0, 0, a:b], q_ref[0, 1, a:b]], axis=2) * scale

    for a, b in chunks:
        asc[a:b] = jnp.concatenate(
            [k_ref[0, 0, 0, a:b], k_ref[0, 0, 1, a:b]], axis=2)
    for dx in range(_K):
        for a, b in chunks:
            kvs[dx, a:b] = _cshift_c(asc, dx, a, b)

    m = jnp.full((_H, _W2), -jnp.inf, dtype=jnp.float32)
    for dy in range(_K):
        sh = _NH - dy
        for a, b in chunks:
            asc[a:b] = _shift_rows2d(qs3[a:b], sh, _H, _W2)

        def p1(dx, m, dy=dy, sh=sh):
            lu = jnp.zeros((_H, _W2), dtype=jnp.float32)
            for a, b in chunks:
                lu = lu + jnp.sum(asc[a:b] * kvs[dx, a:b], axis=0)
            lg = _shift_rows2d(lu, -sh, _H, _W2)
            ls[dy * _K + dx] = lg
            return jnp.maximum(m, lg)

        m = lax.fori_loop(0, _K, p1, m, unroll=True)

    me_all = []
    for si in (0, SI_LAST):
        r0 = si * _TR
        qsv = qs3[:, r0:r0 + _TR, :]

        def e1(o, me, si=si, qsv=qsv, r0=r0):
            dy = o // _K
            dx = o - dy * _K
            kp = _rows_p(kvs, dx, dy, si)
            lg = jnp.sum(qsv * kp, axis=0)
            ls[o, r0:r0 + _TR] = lg
            return jnp.maximum(me, lg)

        me_all.append(lax.fori_loop(
            0, KK, e1, jnp.full((_TR, _W2), -jnp.inf, dtype=jnp.float32),
            unroll=7))

    for a, b in chunks:
        asc[a:b] = jnp.concatenate(
            [v_ref[0, 0, 0, a:b], v_ref[0, 0, 1, a:b]], axis=2)
    for dx in range(_K):
        for a, b in chunks:
            kvs[dx, a:b] = _cshift_c(asc, dx, a, b)

    den = jnp.zeros((_H, _W2), dtype=jnp.float32)
    for a, b in chunks:
        oacc[a:b] = jnp.zeros((CH, _H, _W2), dtype=jnp.float32)
    for dy in range(_K):
        sh = _NH - dy
        for a, b in chunks:
            asc[a:b] = jnp.zeros((CH, _H, _W2), dtype=jnp.float32)

        def p2(dx, den, dy=dy, sh=sh):
            p = jnp.exp(ls[dy * _K + dx] - m)
            pt = _shift_rows2d(p, sh, _H, _W2)
            for a, b in chunks:
                asc[a:b] = asc[a:b] + pt[None] * kvs[dx, a:b]
            return den + p

        den = lax.fori_loop(0, _K, p2, den, unroll=True)
        for a, b in chunks:
            oacc[a:b] = oacc[a:b] + _shift_rows2d(asc[a:b], -sh, _H, _W2)

    inv = pl.reciprocal(den, approx=False)
    for a, b in chunks:
        res = oacc[a:b] * inv[None]
        o_ref[0, 0, a:b] = res[:, :, :_W]
        o_ref[0, 1, a:b] = res[:, :, _W:]

    for si, me in zip((0, SI_LAST), me_all):
        r0 = si * _TR

        def e2(o, carry, si=si, r0=r0, me=me):
            dene, acce = carry
            dy = o // _K
            dx = o - dy * _K
            p = jnp.exp(ls[o, r0:r0 + _TR] - me)
            vp = _rows_p(kvs, dx, dy, si)
            return dene + p, acce + p[None] * vp

        dene, acce = lax.fori_loop(
            0, KK, e2,
            (jnp.zeros((_TR, _W2), dtype=jnp.float32),
             jnp.zeros((_GC, _TR, _W2), dtype=jnp.float32)), unroll=7)

        inve = pl.reciprocal(dene, approx=False)
        rese = acce * inve[None]
        o_ref[0, 0, :, r0:r0 + _TR, :] = rese[:, :, :_W]
        o_ref[0, 1, :, r0:r0 + _TR, :] = rese[:, :, _W:]


def _na2d(q, kv, *, scale):
    B, G = q.shape[0], q.shape[1]
    kern = functools.partial(_na_kernel, scale=scale)
    return pl.pallas_call(
        kern,
        out_shape=jax.ShapeDtypeStruct(q.shape, q.dtype),
        grid=(B, G // 2),
        in_specs=[
            pl.BlockSpec((1, 2, _GC, _H, _W), lambda bi, pi: (bi, pi, 0, 0, 0)),
            pl.BlockSpec((1, 1, 2, _GC, _H, _W),
                         lambda bi, pi: (bi, 0, pi, 0, 0, 0)),
            pl.BlockSpec((1, 1, 2, _GC, _H, _W),
                         lambda bi, pi: (bi, 1, pi, 0, 0, 0)),
        ],
        out_specs=pl.BlockSpec((1, 2, _GC, _H, _W),
                               lambda bi, pi: (bi, pi, 0, 0, 0)),
        scratch_shapes=[
            pltpu.VMEM((_GC, _H, _W2), jnp.float32),
            pltpu.VMEM((_K, _GC, _H, _W2), jnp.float32),
            pltpu.VMEM((_K * _K, _H, _W2), jnp.float32),
            pltpu.VMEM((_GC, _H, _W2), jnp.float32),
            pltpu.VMEM((_GC, _H, _W2), jnp.float32),
        ],
        compiler_params=pltpu.CompilerParams(
            dimension_semantics=("parallel", "parallel")),
    )(q, kv, kv)


def _sample_kernel(x_ref, off_ref, o_ref):
    H, W = _H, _W
    o = off_ref[0, 0]
    offy = jnp.tanh(o[0]) * jnp.float32(1.0 / (H - 1))
    offx = jnp.tanh(o[1]) * jnp.float32(1.0 / (W - 1))
    iy = jax.lax.broadcasted_iota(jnp.int32, (H, W), 0).astype(jnp.float32)
    ix = jax.lax.broadcasted_iota(jnp.int32, (H, W), 1).astype(jnp.float32)
    ref_y = (iy + 0.5) / (H - 1.0) * 2.0 - 1.0
    ref_x = (ix + 0.5) / (W - 1.0) * 2.0 - 1.0
    gy = (offy + ref_y + 1.0) * 0.5 * (H - 1)
    gx = (offx + ref_x + 1.0) * 0.5 * (W - 1)
    wy1 = gy - iy
    wy0 = 1.0 - wy1
    wx1 = gx - ix
    wx0 = 1.0 - wx1

    xx = x_ref[0, 0]
    zc = jnp.zeros((_GC, H, 1), dtype=jnp.float32)
    zr = jnp.zeros((_GC, 1, W), dtype=jnp.float32)
    x_e = jnp.concatenate([xx[:, :, 1:], zc], axis=2)
    x_s = jnp.concatenate([xx[:, 1:, :], zr], axis=1)
    x_se = jnp.concatenate([x_e[:, 1:, :], zr], axis=1)

    out = (xx * (wy0 * wx0)[None] + x_e * (wy0 * wx1)[None]
           + x_s * (wy1 * wx0)[None] + x_se * (wy1 * wx1)[None])
    o_ref[0, 0] = out


def _deform_sample(x_g, off_raw):
    B, G = x_g.shape[0], x_g.shape[1]
    return pl.pallas_call(
        _sample_kernel,
        out_shape=jax.ShapeDtypeStruct(x_g.shape, x_g.dtype),
        grid=(B, G),
        in_specs=[
            pl.BlockSpec((1, 1, _GC, _H, _W), lambda bi, gi: (bi, gi, 0, 0, 0)),
            pl.BlockSpec((1, 1, 2, _H, _W), lambda bi, gi: (bi, gi, 0, 0, 0)),
        ],
        out_specs=pl.BlockSpec((1, 1, _GC, _H, _W),
                               lambda bi, gi: (bi, gi, 0, 0, 0)),
        compiler_params=pltpu.CompilerParams(
            dimension_semantics=("parallel", "parallel")),
    )(x_g, off_raw)


def _depthwise_conv(x, w, b, *, stride=1, padding=0):
    C = x.shape[1]
    y = lax.conv_general_dilated(
        x, w, window_strides=(stride, stride),
        padding=[(padding, padding), (padding, padding)],
        dimension_numbers=("NCHW", "OIHW", "NCHW"),
        feature_group_count=C)
    if b is not None:
        y = y + b[None, :, None, None]
    return y


def _layernorm2d(x, gamma, beta, eps=1e-6):
    u = jnp.mean(x, axis=1, keepdims=True)
    s = jnp.mean((x - u) ** 2, axis=1, keepdims=True)
    xn = (x - u) / jnp.sqrt(s + eps)
    return gamma[None, :, None, None] * xn + beta[None, :, None, None]


def kernel(wq, bq, wk, bk, wv, bv, wo, bo, off_dw_w, off_dw_b,
           off_ln_g, off_ln_b, off_pw_w, rpe_w, rpe_b, x):
    num_heads = 4
    offset_range_factor = 1.0
    B, C, H, W = x.shape
    G = num_heads
    gc = C // G
    scale = gc ** (-0.5)
    K = _K
    S = H * W

    x_flat = x.reshape(B, C, S)

    q = _conv1x1(x_flat, wq, bq)
    q_img = q.reshape(B, C, H, W)

    q_off = q_img.reshape(B * G, gc, H, W)
    t = _depthwise_conv(q_off, off_dw_w, off_dw_b, stride=1, padding=K // 2)
    t = _layernorm2d(t, off_ln_g, off_ln_b)
    t = jax.nn.gelu(t, approximate=False)
    off_raw = jnp.einsum("oc,bchw->bohw", off_pw_w, t)

    x_sampled = _deform_sample(x.reshape(B, G, gc, H, W),
                               off_raw.reshape(B, G, 2, H, W))
    x_sampled = x_sampled.reshape(B, C, S)

    lepe = _depthwise_conv(q_img, rpe_w, rpe_b, stride=1, padding=1)
    lepe_flat = lepe.reshape(B, C, S)

    wkv = jnp.concatenate([wk, wv], axis=0)
    bkv = jnp.concatenate([bk, bv], axis=0)
    kv = _conv1x1(x_sampled, wkv, bkv)

    q_g = q.reshape(B, G, gc, H, W)
    kv_g = kv.reshape(B, 2, G, gc, H, W)
    out = _na2d(q_g, kv_g, scale=scale)
    out = out.reshape(B, C, S)

    y = _conv1x1(out, wo, bo, residual=lepe_flat)
    return y.reshape(B, C, H, W)
